# Initial kernel scaffold; baseline (speedup 1.0000x reference)
import jax
import jax.numpy as jnp
from jax.experimental import pallas as pl


def kernel(x, edge_index, W1, b1, g1, be1, W2, b2, g2, be2, W3, b3, g3, be3):
    raise NotImplementedError("write your pallas kernel here")


# speedup vs baseline: 9.1436x; 9.1436x over previous
import functools
import math

import jax
import jax.numpy as jnp
from jax import lax
from jax.experimental import pallas as pl
from jax.experimental.pallas import tpu as pltpu
from jax.experimental.pallas import tpu_sc as plsc

_EPS = 1e-3
_ISQ = 1.0 / math.sqrt(1.0 + _EPS)

_NC = 2
_NS = 16
_NW = _NC * _NS
_C = 128
_DEGW = 16


@functools.lru_cache(maxsize=None)
def _make_deg_kernel(n_chunks, N, n_pad):
    mesh = plsc.VectorSubcoreMesh(core_axis_name="c", subcore_axis_name="s")
    zr = 64
    rows_per_zero = n_pad // _NS
    n_zero = rows_per_zero // zr
    rows_per_out = n_pad // _NS

    @functools.partial(
        pl.kernel,
        out_type=jax.ShapeDtypeStruct((_NC, n_pad, _DEGW), jnp.float32),
        mesh=mesh,
        scratch_types=[
            pltpu.VMEM((_C,), jnp.int32),
            pltpu.VMEM((_C, _DEGW), jnp.float32),
            pltpu.VMEM((zr, _DEGW), jnp.float32),
            pltpu.VMEM_SHARED((n_pad, _DEGW), jnp.float32),
        ],
    )
    def k(dst_hbm, out_hbm, didx, ones_v, zbuf, acc):
        c = lax.axis_index("c")
        s = lax.axis_index("s")
        wid = c * _NS + s
        one = jnp.ones((16,), jnp.float32)
        zero = jnp.zeros((16,), jnp.float32)

        def fill(i, carry):
            ones_v[i, :] = one
            return carry

        lax.fori_loop(0, _C, fill, 0)

        def zfill(i, carry):
            zbuf[i, :] = zero
            return carry

        lax.fori_loop(0, zr, zfill, 0)

        zbase = s * rows_per_zero

        def zero_body(i, carry):
            pltpu.sync_copy(zbuf, acc.at[pl.ds(zbase + i * zr, zr)])
            return carry

        lax.fori_loop(0, n_zero, zero_body, 0)
        plsc.subcore_barrier()

        ebase = wid * n_chunks * _C

        def body(i, carry):
            pltpu.sync_copy(dst_hbm.at[pl.ds(ebase + i * _C, _C)], didx)
            pltpu.sync_copy(ones_v, acc.at[didx], add=True)
            return carry

        lax.fori_loop(0, n_chunks, body, 0)
        plsc.subcore_barrier()

        r0 = s * rows_per_out
        pltpu.sync_copy(acc.at[pl.ds(r0, rows_per_out)],
                        out_hbm.at[c, pl.ds(r0, rows_per_out)])

    return k


@functools.lru_cache(maxsize=None)
def _make_edge_kernel(n_chunks, N, n_pad, D):
    mesh = plsc.VectorSubcoreMesh(core_axis_name="c", subcore_axis_name="s")
    zr = 8
    rows_per_zero = n_pad // _NS
    n_zero = rows_per_zero // zr
    rows_per_out = n_pad // _NS

    @functools.partial(
        pl.kernel,
        out_type=jax.ShapeDtypeStruct((_NC, n_pad, D), jnp.float32),
        mesh=mesh,
        scratch_types=[
            pltpu.VMEM((_C,), jnp.int32),
            pltpu.VMEM((_C,), jnp.int32),
            pltpu.VMEM((_C, D), jnp.float32),
            pltpu.VMEM((zr, D), jnp.float32),
            pltpu.VMEM_SHARED((n_pad, D), jnp.float32),
            pltpu.SemaphoreType.DMA,
        ],
    )
    def k(src_hbm, dst_hbm, q_hbm, out_hbm, sidx, didx, rows, zbuf, acc, sem):
        c = lax.axis_index("c")
        s = lax.axis_index("s")
        wid = c * _NS + s
        zero = jnp.zeros((16,), jnp.float32)
        for i in range(zr):
            for j in range(D // 16):
                zbuf[i, pl.ds(j * 16, 16)] = zero

        zbase = s * rows_per_zero

        def zero_body(i, carry):
            pltpu.sync_copy(zbuf, acc.at[pl.ds(zbase + i * zr, zr)])
            return carry

        lax.fori_loop(0, n_zero, zero_body, 0)
        plsc.subcore_barrier()

        ebase = wid * n_chunks * _C

        def body(i, carry):
            base = ebase + i * _C
            pltpu.sync_copy(src_hbm.at[pl.ds(base, _C)], sidx)
            pltpu.sync_copy(dst_hbm.at[pl.ds(base, _C)], didx)
            pltpu.async_copy(q_hbm.at[sidx], rows, sem).wait()
            pltpu.sync_copy(rows, acc.at[didx], add=True)
            return carry

        lax.fori_loop(0, n_chunks, body, 0)
        plsc.subcore_barrier()

        r0 = s * rows_per_out
        pltpu.sync_copy(acc.at[pl.ds(r0, rows_per_out)],
                        out_hbm.at[c, pl.ds(r0, rows_per_out)])

    return k


_R = 512


def _first_tc(x, W, dA, dB):
    N, D = x.shape
    grid = pl.cdiv(N, _R)

    def body(x_ref, w_ref, da_ref, db_ref, q_ref, dis_ref):
        dis = lax.rsqrt(da_ref[...] + db_ref[...] + 1.0)
        q_ref[...] = dis * jnp.dot(x_ref[...], w_ref[...],
                                   preferred_element_type=jnp.float32)
        dis_ref[...] = dis

    return pl.pallas_call(
        body,
        grid=(grid,),
        in_specs=[
            pl.BlockSpec((_R, D), lambda i: (i, 0)),
            pl.BlockSpec((D, D), lambda i: (0, 0)),
            pl.BlockSpec((_R, 1), lambda i: (i, 0)),
            pl.BlockSpec((_R, 1), lambda i: (i, 0)),
        ],
        out_specs=[
            pl.BlockSpec((_R, D), lambda i: (i, 0)),
            pl.BlockSpec((_R, 1), lambda i: (i, 0)),
        ],
        out_shape=[
            jax.ShapeDtypeStruct((N, D), jnp.float32),
            jax.ShapeDtypeStruct((N, 1), jnp.float32),
        ],
    )(x, W, dA, dB)


def _mid_tc(s, qp, dis, g, b, be, W):
    N, D = qp.shape

    grid = pl.cdiv(N, _R)

    def body(sa_ref, sb_ref, qp_ref, dis_ref, g_ref, b_ref, be_ref, w_ref,
             q_ref):
        d = dis_ref[...]
        agg = d * (sa_ref[0] + sb_ref[0] + qp_ref[...])
        gs = g_ref[...] * _ISQ
        h = jnp.maximum(gs * agg + (gs * b_ref[...] + be_ref[...]), 0.0)
        q_ref[...] = d * jnp.dot(h, w_ref[...],
                                 preferred_element_type=jnp.float32)

    return pl.pallas_call(
        body,
        grid=(grid,),
        in_specs=[
            pl.BlockSpec((1, _R, D), lambda i: (0, i, 0)),
            pl.BlockSpec((1, _R, D), lambda i: (1, i, 0)),
            pl.BlockSpec((_R, D), lambda i: (i, 0)),
            pl.BlockSpec((_R, 1), lambda i: (i, 0)),
            pl.BlockSpec((1, D), lambda i: (0, 0)),
            pl.BlockSpec((1, D), lambda i: (0, 0)),
            pl.BlockSpec((1, D), lambda i: (0, 0)),
            pl.BlockSpec((D, D), lambda i: (0, 0)),
        ],
        out_specs=pl.BlockSpec((_R, D), lambda i: (i, 0)),
        out_shape=jax.ShapeDtypeStruct((N, D), jnp.float32),
    )(s, s, qp, dis, g, b, be, W)


def _last_tc(s, qp, dis, g, b, be):
    N, D = qp.shape

    grid = pl.cdiv(N, _R)

    def body(sa_ref, sb_ref, qp_ref, dis_ref, g_ref, b_ref, be_ref, o_ref):
        d = dis_ref[...]
        agg = d * (sa_ref[0] + sb_ref[0] + qp_ref[...])
        gs = g_ref[...] * _ISQ
        o_ref[...] = jnp.maximum(gs * agg + (gs * b_ref[...] + be_ref[...]),
                                 0.0)

    return pl.pallas_call(
        body,
        grid=(grid,),
        in_specs=[
            pl.BlockSpec((1, _R, D), lambda i: (0, i, 0)),
            pl.BlockSpec((1, _R, D), lambda i: (1, i, 0)),
            pl.BlockSpec((_R, D), lambda i: (i, 0)),
            pl.BlockSpec((_R, 1), lambda i: (i, 0)),
            pl.BlockSpec((1, D), lambda i: (0, 0)),
            pl.BlockSpec((1, D), lambda i: (0, 0)),
            pl.BlockSpec((1, D), lambda i: (0, 0)),
        ],
        out_specs=pl.BlockSpec((_R, D), lambda i: (i, 0)),
        out_shape=jax.ShapeDtypeStruct((N, D), jnp.float32),
    )(s, s, qp, dis, g, b, be)


def kernel(x, edge_index, W1, b1, g1, be1, W2, b2, g2, be2, W3, b3, g3, be3):
    N, D = x.shape
    E = edge_index.shape[1]
    assert D % 16 == 0 and N % _NS == 0

    n_chunks = pl.cdiv(E, _NW * _C)
    E_pad = n_chunks * _NW * _C
    n_pad = ((N + 1 + 127) // 128) * 128

    src = edge_index[0].astype(jnp.int32)
    dst = edge_index[1].astype(jnp.int32)
    pad = E_pad - E
    if pad:
        src = jnp.concatenate([src, jnp.zeros((pad,), jnp.int32)])
        dst = jnp.concatenate([dst, jnp.full((pad,), N, jnp.int32)])

    deg = _make_deg_kernel(n_chunks, N, n_pad)(dst)
    dA = deg[0, :N, :1]
    dB = deg[1, :N, :1]

    edge = _make_edge_kernel(n_chunks, N, n_pad, D)

    q1, dis = _first_tc(x, W1, dA, dB)
    s1 = edge(src, dst, q1)
    q2 = _mid_tc(s1, q1, dis, g1.reshape(1, D), b1.reshape(1, D),
                 be1.reshape(1, D), W2)
    s2 = edge(src, dst, q2)
    q3 = _mid_tc(s2, q2, dis, g2.reshape(1, D), b2.reshape(1, D),
                 be2.reshape(1, D), W3)
    s3 = edge(src, dst, q3)
    return _last_tc(s3, q3, dis, g3.reshape(1, D), b3.reshape(1, D),
                    be3.reshape(1, D))
